# baseline (speedup 1.0000x reference)
import functools

import jax
import jax.numpy as jnp
from jax import lax
from jax.experimental import pallas as pl
from jax.experimental.pallas import tpu as pltpu
from jax.experimental.pallas import tpu_sc as plsc

VOCAB = 50
BATCH = 16384
SEQ = 200

NW = 32
CPW = BATCH // NW
NG = CPW // 16
NGB = 4
GPB = NG // NGB
REP = 8
PTBL = VOCAB * VOCAB * REP
ROWS_A = 96
ROWS_B = SEQ - ROWS_A


def _build():
    mesh = plsc.VectorSubcoreMesh(core_axis_name="c", subcore_axis_name="s")

    @functools.partial(
        pl.kernel,
        mesh=mesh,
        out_type=jax.ShapeDtypeStruct((BATCH,), jnp.float32),
        compiler_params=pltpu.CompilerParams(needs_layout_passes=False),
        scratch_types=[
            pltpu.VMEM((ROWS_A, CPW), jnp.int32),
            pltpu.VMEM((ROWS_B, CPW), jnp.int32),
            pltpu.VMEM((PTBL,), jnp.float32),
            pltpu.VMEM((CPW,), jnp.float32),
            pltpu.SemaphoreType.DMA,
            pltpu.SemaphoreType.DMA,
        ],
    )
    def k(idxt_hbm, ptable_hbm, out_hbm, buf_a, buf_b, ptable_v, acc_v,
          sem_a, sem_b):
        wid = lax.axis_index("s") * 2 + lax.axis_index("c")
        col0 = wid * CPW
        pltpu.async_copy(
            idxt_hbm.at[pl.ds(0, ROWS_A), pl.ds(col0, CPW)], buf_a, sem_a
        )
        pltpu.async_copy(
            idxt_hbm.at[pl.ds(ROWS_A, ROWS_B), pl.ds(col0, CPW)], buf_b, sem_b
        )
        pltpu.sync_copy(ptable_hbm, ptable_v)
        lane_rep = lax.iota(jnp.int32, 16) & (REP - 1)

        def run_bands(buf, nbands, gb, accs):
            @plsc.parallel_loop(0, nbands, unroll=1, carry=tuple(accs))
            def band_body(t, accs):
                row0 = t * 8
                accs = list(accs)
                for r in range(0, 8, 2):
                    for j in range(GPB):
                        g = gb * GPB + j
                        iv1 = buf[row0 + r, pl.ds(16 * g, 16)]
                        iv2 = buf[row0 + r + 1, pl.ds(16 * g, 16)]
                        cidx = (iv1 * (VOCAB * REP) + iv2 * REP) + lane_rep
                        val = plsc.load_gather(ptable_v, [cidx])
                        accs[j] = accs[j] + val
                return tuple(accs)

            return band_body

        zero = jnp.zeros((16,), jnp.float32)
        pltpu.make_async_copy(
            idxt_hbm.at[pl.ds(0, ROWS_A), pl.ds(col0, CPW)], buf_a, sem_a
        ).wait()
        for gb in range(NGB):
            accs = run_bands(buf_a, ROWS_A // 8, gb,
                             tuple(zero for _ in range(GPB)))
            for j in range(GPB):
                acc_v[pl.ds(16 * (gb * GPB + j), 16)] = accs[j]
        pltpu.make_async_copy(
            idxt_hbm.at[pl.ds(ROWS_A, ROWS_B), pl.ds(col0, CPW)], buf_b, sem_b
        ).wait()
        for gb in range(NGB):
            accs = run_bands(
                buf_b, ROWS_B // 8, gb,
                tuple(acc_v[pl.ds(16 * (gb * GPB + j), 16)]
                      for j in range(GPB)),
            )
            for j in range(GPB):
                acc_v[pl.ds(16 * (gb * GPB + j), 16)] = accs[j]
        pltpu.sync_copy(acc_v, out_hbm.at[pl.ds(col0, CPW)])

    return k


_sc_kernel = _build()


@jax.jit
def kernel(inputs, tag_table):
    ptable = (tag_table[:, None] + tag_table[None, :]).reshape(-1)
    ptable_rep = jnp.repeat(ptable, REP)
    return _sc_kernel(inputs.T, ptable_rep)


# speedup vs baseline: 1.0648x; 1.0648x over previous
import functools

import jax
import jax.numpy as jnp
from jax import lax
from jax.experimental import pallas as pl
from jax.experimental.pallas import tpu as pltpu
from jax.experimental.pallas import tpu_sc as plsc

VOCAB = 50
BATCH = 16384
SEQ = 200

NW = 32
CPW = BATCH // NW
NG = CPW // 16
NGB = 4
GPB = NG // NGB
PTBL = 2560
ROWS_A = 64
ROWS_B = SEQ - ROWS_A


def _build():
    mesh = plsc.VectorSubcoreMesh(core_axis_name="c", subcore_axis_name="s")

    @functools.partial(
        pl.kernel,
        mesh=mesh,
        out_type=jax.ShapeDtypeStruct((BATCH,), jnp.float32),
        compiler_params=pltpu.CompilerParams(needs_layout_passes=False),
        scratch_types=[
            pltpu.VMEM((ROWS_A, CPW), jnp.int32),
            pltpu.VMEM((ROWS_B, CPW), jnp.int32),
            pltpu.VMEM((PTBL,), jnp.float32),
            pltpu.VMEM((CPW,), jnp.float32),
            pltpu.SemaphoreType.DMA,
            pltpu.SemaphoreType.DMA,
        ],
    )
    def k(idxt_hbm, ptable_hbm, out_hbm, buf_a, buf_b, ptable_v, acc_v,
          sem_a, sem_b):
        wid = lax.axis_index("s") * 2 + lax.axis_index("c")
        col0 = wid * CPW
        pltpu.async_copy(
            idxt_hbm.at[pl.ds(0, ROWS_A), pl.ds(col0, CPW)], buf_a, sem_a
        )
        pltpu.async_copy(
            idxt_hbm.at[pl.ds(ROWS_A, ROWS_B), pl.ds(col0, CPW)], buf_b, sem_b
        )
        pltpu.sync_copy(ptable_hbm, ptable_v)

        def run_bands(buf, nbands, gb, accs):
            @plsc.parallel_loop(0, nbands, unroll=1, carry=tuple(accs))
            def band_body(t, accs):
                row0 = t * 8
                accs = list(accs)
                for r in range(0, 8, 2):
                    for j in range(GPB):
                        g = gb * GPB + j
                        iv1 = buf[row0 + r, pl.ds(16 * g, 16)]
                        iv2 = buf[row0 + r + 1, pl.ds(16 * g, 16)]
                        cidx = iv1 * VOCAB + iv2
                        val = plsc.load_gather(ptable_v, [cidx])
                        accs[j] = accs[j] + val
                return tuple(accs)

            return band_body

        zero = jnp.zeros((16,), jnp.float32)
        pltpu.make_async_copy(
            idxt_hbm.at[pl.ds(0, ROWS_A), pl.ds(col0, CPW)], buf_a, sem_a
        ).wait()
        for gb in range(NGB):
            accs = run_bands(buf_a, ROWS_A // 8, gb,
                             tuple(zero for _ in range(GPB)))
            for j in range(GPB):
                acc_v[pl.ds(16 * (gb * GPB + j), 16)] = accs[j]
        pltpu.make_async_copy(
            idxt_hbm.at[pl.ds(ROWS_A, ROWS_B), pl.ds(col0, CPW)], buf_b, sem_b
        ).wait()
        for gb in range(NGB):
            accs = run_bands(
                buf_b, ROWS_B // 8, gb,
                tuple(acc_v[pl.ds(16 * (gb * GPB + j), 16)]
                      for j in range(GPB)),
            )
            for j in range(GPB):
                acc_v[pl.ds(16 * (gb * GPB + j), 16)] = accs[j]
        pltpu.sync_copy(acc_v, out_hbm.at[pl.ds(col0, CPW)])

    return k


_sc_kernel = _build()


@jax.jit
def kernel(inputs, tag_table):
    ptable = (tag_table[:, None] + tag_table[None, :]).reshape(-1)
    ptable = jnp.concatenate(
        [ptable, jnp.zeros((PTBL - VOCAB * VOCAB,), jnp.float32)]
    )
    return _sc_kernel(inputs.T, ptable)
